# Initial kernel scaffold; baseline (speedup 1.0000x reference)
import jax
import jax.numpy as jnp
from jax.experimental import pallas as pl


def kernel(x, edge_index, W1, b1, W2, b2, W3, b3):
    raise NotImplementedError("write your pallas kernel here")


# speedup vs baseline: 5.1076x; 5.1076x over previous
import functools

import jax
import jax.numpy as jnp
from jax import lax
from jax.experimental import pallas as pl
from jax.experimental.pallas import tpu as pltpu
from jax.experimental.pallas import tpu_sc as plsc

N = 10000
D = 128
NC = 2
NS = 16
NT = NC * NS

EBd = 80
NP = 10240
RPTd = NP // NS

EB = 160
EPAD = NS * EB * 128
HALF = 5120
NDUMP = 256
NPa = HALF + NDUMP
RPT = NPa // NS

_mesh = plsc.VectorSubcoreMesh(core_axis_name="c", subcore_axis_name="s")


@functools.partial(
    pl.kernel,
    out_type=jax.ShapeDtypeStruct((NC, NP), jnp.float32),
    mesh=_mesh,
    scratch_types=[
        pltpu.VMEM((EBd, 128), jnp.int32),
        pltpu.VMEM((128,), jnp.float32),
        pltpu.VMEM((RPTd,), jnp.float32),
        pltpu.VMEM_SHARED((NP,), jnp.float32),
    ],
)
def _degree_kernel(dst_hbm, deg_hbm, dstv, onesv, zv, acc):
    c = lax.axis_index("c")
    s = lax.axis_index("s")
    g = c * NS + s
    for k in range(8):
        onesv[pl.ds(k * 16, 16)] = jnp.ones((16,), jnp.float32)
    for k in range(RPTd // 16):
        zv[pl.ds(k * 16, 16)] = jnp.zeros((16,), jnp.float32)
    pltpu.sync_copy(zv, acc.at[pl.ds(s * RPTd, RPTd)])
    pltpu.sync_copy(dst_hbm.at[pl.ds(g * EBd, EBd)], dstv)
    plsc.subcore_barrier()
    for j in range(EBd):
        pltpu.sync_copy(onesv, acc.at[dstv.at[j]], add=True)
    plsc.subcore_barrier()
    pltpu.sync_copy(acc.at[pl.ds(s * RPTd, RPTd)],
                    deg_hbm.at[c, pl.ds(s * RPTd, RPTd)])


@functools.partial(
    pl.kernel,
    out_type=jax.ShapeDtypeStruct((NC, NPa, D), jnp.float32),
    mesh=_mesh,
    scratch_types=[
        pltpu.VMEM((EB, 128), jnp.int32),
        pltpu.VMEM((EB, 128), jnp.int32),
        pltpu.VMEM((128, D), jnp.float32),
        pltpu.VMEM((128, D), jnp.float32),
        pltpu.VMEM_SHARED((NPa, D), jnp.float32),
        pltpu.SemaphoreType.DMA,
        pltpu.SemaphoreType.DMA,
    ],
)
def _agg_kernel(h_hbm, src_hbm, dst_hbm, zeros_hbm, out_hbm,
                srcv, dstv, rows0, rows1, acc, sem0, sem1):
    c = lax.axis_index("c")
    s = lax.axis_index("s")
    pltpu.sync_copy(zeros_hbm, acc.at[pl.ds(s * RPT, RPT)])
    pltpu.sync_copy(src_hbm.at[pl.ds(s * EB, EB)], srcv)
    pltpu.sync_copy(dst_hbm.at[c].at[pl.ds(s * EB, EB)], dstv)
    plsc.subcore_barrier()
    bufs = (rows0, rows1)
    sems = (sem0, sem1)
    desc = [None, None]
    desc[0] = pltpu.async_copy(h_hbm.at[srcv.at[0]], bufs[0], sems[0])
    for j in range(EB):
        p = j % 2
        if j + 1 < EB:
            q = (j + 1) % 2
            desc[q] = pltpu.async_copy(h_hbm.at[srcv.at[j + 1]], bufs[q], sems[q])
        desc[p].wait()
        pltpu.sync_copy(bufs[p], acc.at[dstv.at[j]], add=True)
    plsc.subcore_barrier()
    pltpu.sync_copy(acc.at[pl.ds(s * RPT, RPT)],
                    out_hbm.at[c].at[pl.ds(s * RPT, RPT)])


R = 1000


def _mm_first_body(x_ref, w_ref, d0_ref, d1_ref, hp_ref, dinv_ref):
    dinv = lax.rsqrt(d0_ref[...] + d1_ref[...] + 1.0)
    hp_ref[...] = jnp.dot(x_ref[...], w_ref[...],
                          preferred_element_type=jnp.float32) * dinv
    dinv_ref[...] = dinv


def _mm_mid_body(p_ref, hp_ref, dinv_ref, b_ref, w_ref, out_ref):
    dinv = dinv_ref[...]
    t = (p_ref[...] + hp_ref[...]) * dinv + b_ref[...]
    t = jnp.maximum(t, 0.0)
    out_ref[...] = jnp.dot(t, w_ref[...],
                           preferred_element_type=jnp.float32) * dinv


def _final_body(p_ref, hp_ref, dinv_ref, b_ref, out_ref):
    z = (p_ref[...] + hp_ref[...]) * dinv_ref[...] + b_ref[...]
    m = jnp.max(z, axis=1, keepdims=True)
    lse = jnp.log(jnp.sum(jnp.exp(z - m), axis=1, keepdims=True))
    out_ref[...] = z - m - lse


def _row_spec(width):
    return pl.BlockSpec((R, width), lambda i: (i, 0))


def _full_spec(shape):
    return pl.BlockSpec(shape, lambda i: tuple(0 for _ in shape))


def _mm_first(x, w, d0, d1):
    return pl.pallas_call(
        _mm_first_body,
        grid=(N // R,),
        in_specs=[_row_spec(D), _full_spec((D, D)), _row_spec(1), _row_spec(1)],
        out_specs=[_row_spec(D), _row_spec(1)],
        out_shape=[jax.ShapeDtypeStruct((N, D), jnp.float32),
                   jax.ShapeDtypeStruct((N, 1), jnp.float32)],
    )(x, w, d0, d1)


def _mm_mid(p, hp, dinv, b, w):
    return pl.pallas_call(
        _mm_mid_body,
        grid=(N // R,),
        in_specs=[_row_spec(D), _row_spec(D), _row_spec(1),
                  _full_spec((1, D)), _full_spec((D, D))],
        out_specs=_row_spec(D),
        out_shape=jax.ShapeDtypeStruct((N, D), jnp.float32),
    )(p, hp, dinv, b, w)


def _final(p, hp, dinv, b):
    return pl.pallas_call(
        _final_body,
        grid=(N // R,),
        in_specs=[_row_spec(D), _row_spec(D), _row_spec(1),
                  _full_spec((1, D))],
        out_specs=_row_spec(D),
        out_shape=jax.ShapeDtypeStruct((N, D), jnp.float32),
    )(p, hp, dinv, b)


def kernel(x, edge_index, W1, b1, W2, b2, W3, b3):
    src = edge_index[0]
    dst = edge_index[1]
    pad = EPAD - src.shape[0]
    src_p = jnp.concatenate([src, jnp.zeros((pad,), jnp.int32)])
    dst_p = jnp.concatenate([dst, jnp.full((pad,), -1, jnp.int32)])
    src2d = src_p.reshape(NS * EB, 128)
    dst2d_deg = jnp.where(dst_p < 0, N, dst_p).reshape(NS * EB, 128)
    dump = HALF + (jnp.arange(EPAD, dtype=jnp.int32) % NDUMP)
    halves = []
    for c in range(NC):
        loc = dst_p - c * HALF
        own = (loc >= 0) & (loc < HALF)
        halves.append(jnp.where(own, loc, dump))
    dst3d = jnp.stack(halves).reshape(NC, NS * EB, 128)
    zeros = jnp.zeros((RPT, D), jnp.float32)

    degp = _degree_kernel(dst2d_deg)
    d0 = degp[0].reshape(NP, 1)
    d1 = degp[1].reshape(NP, 1)

    h1p, dinv = _mm_first(x, W1, d0[:N], d1[:N])
    P = _agg_kernel(h1p, src2d, dst3d, zeros)
    S = jnp.concatenate([P[0, :HALF], P[1, :N - HALF]])
    h2p = _mm_mid(S, h1p, dinv, b1.reshape(1, D), W2)
    P = _agg_kernel(h2p, src2d, dst3d, zeros)
    S = jnp.concatenate([P[0, :HALF], P[1, :N - HALF]])
    h3p = _mm_mid(S, h2p, dinv, b2.reshape(1, D), W3)
    P = _agg_kernel(h3p, src2d, dst3d, zeros)
    S = jnp.concatenate([P[0, :HALF], P[1, :N - HALF]])
    return _final(S, h3p, dinv, b3.reshape(1, D))
